# baseline (speedup 1.0000x reference)
import functools

import jax
import jax.numpy as jnp
from jax import lax
from jax.experimental import pallas as pl
from jax.experimental.pallas import tpu as pltpu
from jax.experimental.pallas import tpu_sc as plsc

B = 16384
D = 64
C = 5
NW = 32
BPW = B // NW
CHUNK = 128
NCHUNK = BPW // CHUNK


def _w2v_body(ttab, ctab, tidx_hbm, cidx_hbm, out_hbm,
              tidx_v, cidx_v, trows_v, crows_v, out_v, sem):
    wid = lax.axis_index("s") * 2 + lax.axis_index("c")

    for chunk in range(NCHUNK):
        row = wid * NCHUNK + chunk
        base = row * CHUNK

        pltpu.sync_copy(tidx_hbm.at[pl.ds(base, CHUNK)], tidx_v)
        pltpu.sync_copy(cidx_hbm.at[pl.ds(base * C, CHUNK * C)], cidx_v)

        cps = [pltpu.async_copy(ttab.at[tidx_v], trows_v, sem)]
        for j in range(C):
            cps.append(pltpu.async_copy(
                ctab.at[cidx_v.at[pl.ds(j * CHUNK, CHUNK)]],
                crows_v.at[pl.ds(j * CHUNK, CHUNK)], sem))
        for cp in cps:
            cp.wait()

        lane15 = jnp.arange(16, dtype=jnp.int32) == 15

        def body(g, _):
            for bl in range(16):
                i = g * 16 + bl
                t = [trows_v[i, pl.ds(16 * k, 16)] for k in range(4)]
                for c in range(C):
                    r = i * C + c
                    acc = t[0] * crows_v[r, pl.ds(0, 16)]
                    for k in range(1, 4):
                        acc = acc + t[k] * crows_v[r, pl.ds(16 * k, 16)]
                    cums = jnp.cumsum(acc)
                    plsc.store_scatter(
                        out_v, [jnp.full((16,), r, jnp.int32)], cums,
                        mask=lane15)
            return 0

        lax.fori_loop(0, CHUNK // 16, body, 0)

        pltpu.sync_copy(out_v, out_hbm.at[pl.ds(base * C, CHUNK * C)])


def kernel(target, context, target_table, context_table):
    mesh = plsc.VectorSubcoreMesh(core_axis_name="c", subcore_axis_name="s")
    ctx_flat = context.reshape(B * C).astype(jnp.int32)
    tgt = target.astype(jnp.int32)

    run = functools.partial(
        pl.kernel,
        mesh=mesh,
        compiler_params=pltpu.CompilerParams(
            needs_layout_passes=False, use_tc_tiling_on_sc=False),
        out_type=jax.ShapeDtypeStruct((B * C,), jnp.float32),
        scratch_types=[
            pltpu.VMEM((CHUNK,), jnp.int32),
            pltpu.VMEM((CHUNK * C,), jnp.int32),
            pltpu.VMEM((CHUNK, D), jnp.float32),
            pltpu.VMEM((CHUNK * C, D), jnp.float32),
            pltpu.VMEM((CHUNK * C,), jnp.float32),
            pltpu.SemaphoreType.DMA,
        ],
    )(_w2v_body)

    out = run(target_table, context_table, tgt, ctx_flat)
    return out.reshape(B, C)
